# Initial kernel scaffold; baseline (speedup 1.0000x reference)
import jax
import jax.numpy as jnp
from jax.experimental import pallas as pl


def kernel(emb, src_org, pe):
    raise NotImplementedError("write your pallas kernel here")


# speedup vs baseline: 1.3023x; 1.3023x over previous
import math

import jax
import jax.numpy as jnp
from jax.experimental import pallas as pl


def _pe_add_block(emb_ref, pe_ref, out_ref, *, scale):
    out_ref[...] = emb_ref[...] * scale + pe_ref[...]


def kernel(emb, src_org, pe):
    del src_org
    seq, b, dim = emb.shape
    scale = math.sqrt(pe.shape[-1])

    block_s = 256
    grid = (seq // block_s,)

    return pl.pallas_call(
        lambda e, p, o: _pe_add_block(e, p, o, scale=scale),
        grid=grid,
        in_specs=[
            pl.BlockSpec((block_s, b, dim), lambda i: (i, 0, 0)),
            pl.BlockSpec((block_s, 1, dim), lambda i: (i, 0, 0)),
        ],
        out_specs=pl.BlockSpec((block_s, b, dim), lambda i: (i, 0, 0)),
        out_shape=jax.ShapeDtypeStruct((seq, b, dim), emb.dtype),
    )(emb, pe[:seq])
